# baseline (speedup 1.0000x reference)
import functools

import jax
import jax.numpy as jnp
from jax.experimental import pallas as pl
from jax.experimental.pallas import tpu as pltpu
from jax.experimental.pallas import tpu_sc as plsc

_BATCH = 8
_D_IN = 64
_VOCAB = 1_000_000

_BLOCK_V = 32768

_NC = 2
_NS = 16
_SC_COLS = 98304
_W_PER = _SC_COLS // (_NC * _NS)
_CHUNK = 768
_N_CHUNKS = _W_PER // _CHUNK
_JB = 4


def _tc_body(state_ref, w_ref, out_ref):
    out_ref[...] = jnp.dot(
        state_ref[...], w_ref[...], preferred_element_type=jnp.float32
    )


def _tc_matmul(state, W):
    n_blocks = pl.cdiv(_VOCAB - _SC_COLS, _BLOCK_V)
    off = _SC_COLS // _BLOCK_V
    return pl.pallas_call(
        _tc_body,
        grid=(n_blocks,),
        in_specs=[
            pl.BlockSpec((_BATCH, _D_IN), lambda i: (0, 0)),
            pl.BlockSpec((_D_IN, _BLOCK_V), lambda i: (0, i + off)),
        ],
        out_specs=pl.BlockSpec((_BATCH, _BLOCK_V), lambda i: (0, i + off)),
        out_shape=jax.ShapeDtypeStruct((_BATCH, _VOCAB), jnp.float32),
        compiler_params=pltpu.CompilerParams(
            dimension_semantics=("parallel",),
        ),
    )(state, W)


def _sc_body(state_hbm, w_hbm, out_hbm, state_v, wbuf, outbuf):
    cid = jax.lax.axis_index("c")
    sid = jax.lax.axis_index("s")
    wid = sid * _NC + cid
    base = wid * _W_PER
    pltpu.sync_copy(state_hbm, state_v)

    def chunk_body(ci, carry):
        col0 = base + ci * _CHUNK
        pltpu.sync_copy(w_hbm.at[:, pl.ds(col0, _CHUNK)], wbuf)

        def jb_body(jb, inner_carry):
            c = jb * (_JB * 16)
            accs = [
                [jnp.zeros((16,), jnp.float32) for _ in range(_JB)]
                for _ in range(_BATCH)
            ]
            for kb in range(_D_IN // 16):
                sv = [state_v[b, pl.ds(kb * 16, 16)] for b in range(_BATCH)]
                for kk in range(16):
                    k = kb * 16 + kk
                    wv = [wbuf[k, pl.ds(c + 16 * jj, 16)] for jj in range(_JB)]
                    for b in range(_BATCH):
                        s = sv[b][kk]
                        for jj in range(_JB):
                            accs[b][jj] = accs[b][jj] + s * wv[jj]
            for b in range(_BATCH):
                for jj in range(_JB):
                    outbuf[b, pl.ds(ci * _CHUNK + c + 16 * jj, 16)] = accs[b][jj]
            return inner_carry

        jax.lax.fori_loop(0, _CHUNK // (_JB * 16), jb_body, 0)
        return carry

    jax.lax.fori_loop(0, _N_CHUNKS, chunk_body, 0)
    pltpu.sync_copy(outbuf, out_hbm.at[:, pl.ds(base, _W_PER)])


@functools.partial(jax.jit, static_argnums=())
def _sc_matmul(state, W):
    mesh = plsc.VectorSubcoreMesh(core_axis_name="c", subcore_axis_name="s")
    f = functools.partial(
        pl.kernel,
        out_type=jax.ShapeDtypeStruct((_BATCH, _SC_COLS), jnp.float32),
        mesh=mesh,
        scratch_types=[
            pltpu.VMEM((_BATCH, _D_IN), jnp.float32),
            pltpu.VMEM((_D_IN, _CHUNK), jnp.float32),
            pltpu.VMEM((_BATCH, _W_PER), jnp.float32),
        ],
        compiler_params=pltpu.CompilerParams(use_tc_tiling_on_sc=True),
    )(_sc_body)
    return f(state, W)


def kernel(state, W):
    tc_out = _tc_matmul(state, W)
    sc_out = _sc_matmul(state, W)
    return jax.lax.dynamic_update_slice(tc_out, sc_out, (0, 0))


# speedup vs baseline: 1.3187x; 1.3187x over previous
import functools

import jax
import jax.numpy as jnp
from jax.experimental import pallas as pl
from jax.experimental.pallas import tpu as pltpu
from jax.experimental.pallas import tpu_sc as plsc

_BATCH = 8
_D_IN = 64
_VOCAB = 1_000_000

_BLOCK_V = 32768

_NC = 2
_NS = 16
_SC_COLS = 65536
_W_PER = _SC_COLS // (_NC * _NS)
_CHUNK = 512
_N_CHUNKS = _W_PER // _CHUNK
_JB = 4


def _tc_body(state_ref, w_ref, out_ref):
    out_ref[...] = jnp.dot(
        state_ref[...], w_ref[...], preferred_element_type=jnp.float32
    )


def _tc_matmul(state, W):
    n_blocks = pl.cdiv(_VOCAB - _SC_COLS, _BLOCK_V)
    off = _SC_COLS // _BLOCK_V
    return pl.pallas_call(
        _tc_body,
        grid=(n_blocks,),
        in_specs=[
            pl.BlockSpec((_BATCH, _D_IN), lambda i: (0, 0)),
            pl.BlockSpec((_D_IN, _BLOCK_V), lambda i: (0, i + off)),
        ],
        out_specs=pl.BlockSpec((_BATCH, _BLOCK_V), lambda i: (0, i + off)),
        out_shape=jax.ShapeDtypeStruct((_BATCH, _VOCAB), jnp.float32),
        compiler_params=pltpu.CompilerParams(
            dimension_semantics=("parallel",),
        ),
    )(state, W)


def _sc_body(state_hbm, w_hbm, out_hbm, state_v, wbuf, outbuf):
    cid = jax.lax.axis_index("c")
    sid = jax.lax.axis_index("s")
    wid = sid * _NC + cid
    base = wid * _W_PER
    pltpu.sync_copy(state_hbm, state_v)

    def chunk_body(ci, carry):
        col0 = base + ci * _CHUNK
        pltpu.sync_copy(w_hbm.at[:, pl.ds(col0, _CHUNK)], wbuf)

        def jb_body(jb, inner_carry):
            c = jb * (_JB * 16)
            accs = [
                [jnp.zeros((16,), jnp.float32) for _ in range(_JB)]
                for _ in range(_BATCH)
            ]
            for kb in range(_D_IN // 16):
                sv = [state_v[b, pl.ds(kb * 16, 16)] for b in range(_BATCH)]
                for kk in range(16):
                    k = kb * 16 + kk
                    wv = [wbuf[k, pl.ds(c + 16 * jj, 16)] for jj in range(_JB)]
                    for b in range(_BATCH):
                        s = sv[b][kk]
                        for jj in range(_JB):
                            accs[b][jj] = accs[b][jj] + s * wv[jj]
            for b in range(_BATCH):
                for jj in range(_JB):
                    outbuf[b, pl.ds(ci * _CHUNK + c + 16 * jj, 16)] = accs[b][jj]
            return inner_carry

        jax.lax.fori_loop(0, _CHUNK // (_JB * 16), jb_body, 0)
        return carry

    jax.lax.fori_loop(0, _N_CHUNKS, chunk_body, 0)
    pltpu.sync_copy(outbuf, out_hbm.at[:, pl.ds(base, _W_PER)])


def _sc_matmul(state, W):
    mesh = plsc.VectorSubcoreMesh(core_axis_name="c", subcore_axis_name="s")
    f = functools.partial(
        pl.kernel,
        out_type=jax.ShapeDtypeStruct((_BATCH, _SC_COLS), jnp.float32),
        mesh=mesh,
        scratch_types=[
            pltpu.VMEM((_BATCH, _D_IN), jnp.float32),
            pltpu.VMEM((_D_IN, _CHUNK), jnp.float32),
            pltpu.VMEM((_BATCH, _W_PER), jnp.float32),
        ],
        compiler_params=pltpu.CompilerParams(use_tc_tiling_on_sc=True),
    )(_sc_body)
    return f(state, W)


def _merge_body(tc_ref, sc_ref, out_ref):
    out_ref[...] = sc_ref[...]


def _merge(tc_out, sc_out):
    return pl.pallas_call(
        _merge_body,
        grid=(_SC_COLS // _BLOCK_V,),
        in_specs=[
            pl.BlockSpec(memory_space=pl.ANY),
            pl.BlockSpec((_BATCH, _BLOCK_V), lambda i: (0, i)),
        ],
        out_specs=pl.BlockSpec((_BATCH, _BLOCK_V), lambda i: (0, i)),
        out_shape=jax.ShapeDtypeStruct((_BATCH, _VOCAB), jnp.float32),
        input_output_aliases={0: 0},
    )(tc_out, sc_out)


def kernel(state, W):
    tc_out = _tc_matmul(state, W)
    sc_out = _sc_matmul(state, W)
    return _merge(tc_out, sc_out)


# speedup vs baseline: 1.7119x; 1.2982x over previous
import jax
import jax.numpy as jnp
from jax.experimental import pallas as pl
from jax.experimental.pallas import tpu as pltpu

_BATCH = 8
_D_IN = 64
_VOCAB = 1_000_000
_BLOCK_V = 40960


def _matmul_body(state_ref, w_ref, out_ref):
    out_ref[...] = jnp.dot(
        state_ref[...], w_ref[...], preferred_element_type=jnp.float32
    )


def kernel(state, W):
    grid = pl.cdiv(_VOCAB, _BLOCK_V)
    return pl.pallas_call(
        _matmul_body,
        grid=(grid,),
        in_specs=[
            pl.BlockSpec((_BATCH, _D_IN), lambda i: (0, 0)),
            pl.BlockSpec((_D_IN, _BLOCK_V), lambda i: (0, i)),
        ],
        out_specs=pl.BlockSpec((_BATCH, _BLOCK_V), lambda i: (0, i)),
        out_shape=jax.ShapeDtypeStruct((_BATCH, _VOCAB), jnp.float32),
        compiler_params=pltpu.CompilerParams(
            dimension_semantics=("parallel",),
        ),
    )(state, W)
